# Initial kernel scaffold; baseline (speedup 1.0000x reference)
import functools

import jax
import jax.numpy as jnp
from jax import lax
from jax.experimental import pallas as pl
from jax.experimental.pallas import tpu as pltpu
from jax.experimental.pallas import tpu_sc as plsc

N = 10000
E = 160000
D = 256
H = 128
NCORE = 2
NSUB = 16
CH = 128
EPT = 10240
EPAD = EPT * NSUB
NCHUNK = EPT // CH
NPAD = N + 16
RPT = N // NSUB


def _segsum_body(with_counts, *refs):
    if with_counts:
        (x_hbm, src_hbm, dst_hbm, zf_hbm, zc_hbm, ones_hbm,
         sum_hbm, cnt_hbm,
         src_v, dst_v, rows_v, ones_v, acc_sh, cnt_sh, sem) = refs
    else:
        (x_hbm, src_hbm, dst_hbm, zf_hbm,
         sum_hbm,
         src_v, dst_v, rows_v, acc_sh, sem) = refs
    c = lax.axis_index("c")
    s = lax.axis_index("s")

    pltpu.sync_copy(zf_hbm, acc_sh.at[pl.ds(s * RPT, RPT)])
    if with_counts:
        pltpu.sync_copy(zc_hbm, cnt_sh.at[pl.ds(s * RPT, RPT)])
        pltpu.sync_copy(ones_hbm, ones_v)
    plsc.subcore_barrier()

    def chunk(k, carry):
        e0 = c * EPAD + s * EPT + k * CH
        pltpu.sync_copy(src_hbm.at[pl.ds(e0, CH)], src_v)
        pltpu.sync_copy(dst_hbm.at[pl.ds(s * EPT + k * CH, CH)], dst_v)
        pltpu.async_copy(x_hbm.at[src_v], rows_v, sem).wait()
        pltpu.sync_copy(rows_v, acc_sh.at[dst_v], add=True)
        if with_counts:
            pltpu.sync_copy(ones_v, cnt_sh.at[dst_v], add=True)
        return carry

    lax.fori_loop(0, NCHUNK, chunk, 0)
    plsc.subcore_barrier()

    pltpu.sync_copy(acc_sh.at[pl.ds(s * RPT, RPT)],
                    sum_hbm.at[pl.ds(c * N + s * RPT, RPT)])
    if with_counts:
        @pl.when(c == 0)
        def _():
            pltpu.sync_copy(cnt_sh.at[pl.ds(s * RPT, RPT)],
                            cnt_hbm.at[pl.ds(s * RPT, RPT)])


def _make_segsum(with_counts):
    mesh = plsc.VectorSubcoreMesh(core_axis_name="c", subcore_axis_name="s")
    out_type = [jax.ShapeDtypeStruct((NCORE * N, H), jnp.float32)]
    if with_counts:
        out_type.append(jax.ShapeDtypeStruct((N, 16), jnp.float32))
    scratch = [
        pltpu.VMEM((CH,), jnp.int32),
        pltpu.VMEM((CH,), jnp.int32),
        pltpu.VMEM((CH, H), jnp.float32),
    ]
    if with_counts:
        scratch.append(pltpu.VMEM((CH, 16), jnp.float32))
    scratch.append(pltpu.VMEM_SHARED((NPAD, H), jnp.float32))
    if with_counts:
        scratch.append(pltpu.VMEM_SHARED((NPAD, 16), jnp.float32))
    scratch.append(pltpu.SemaphoreType.DMA)
    return pl.kernel(
        functools.partial(_segsum_body, with_counts),
        out_type=tuple(out_type),
        mesh=mesh,
        scratch_types=scratch,
    )


def _tc1_body(sum_ref, x_ref, cnt_ref, a_ref, b_ref, bias_ref, out_ref):
    inv = 1.0 / jnp.maximum(cnt_ref[...], 1.0)
    m = jnp.concatenate([sum_ref[0], sum_ref[1]], axis=1) * inv
    xc = jnp.concatenate([x_ref[0], x_ref[1]], axis=1)
    h = (jnp.dot(m, a_ref[...], preferred_element_type=jnp.float32)
         + jnp.dot(xc, b_ref[...], preferred_element_type=jnp.float32)
         + bias_ref[...])
    h = jnp.maximum(h, 0.0)
    out_ref[0] = h[:, :H]
    out_ref[1] = h[:, H:]


def _tc2_body(sum_ref, h_ref, cnt_ref, a_ref, b_ref, bias_ref, out_ref):
    inv = 1.0 / jnp.maximum(cnt_ref[...], 1.0)
    m = jnp.concatenate([sum_ref[0], sum_ref[1]], axis=1) * inv
    hc = jnp.concatenate([h_ref[0], h_ref[1]], axis=1)
    g = (jnp.dot(m, a_ref[...], preferred_element_type=jnp.float32)
         + jnp.dot(hc, b_ref[...], preferred_element_type=jnp.float32)
         + bias_ref[...])
    g = jnp.maximum(g, 0.0)
    ss = jnp.sum(g * g, axis=1, keepdims=True)
    out_ref[...] = g / jnp.maximum(jnp.sqrt(ss), 1e-12)


_R = 1000


def _tc1(summed, xs, cnt, A, B, bias):
    return pl.pallas_call(
        _tc1_body,
        grid=(N // _R,),
        in_specs=[
            pl.BlockSpec((NCORE, _R, H), lambda i: (0, i, 0)),
            pl.BlockSpec((NCORE, _R, H), lambda i: (0, i, 0)),
            pl.BlockSpec((_R, 1), lambda i: (i, 0)),
            pl.BlockSpec((D, D), lambda i: (0, 0)),
            pl.BlockSpec((D, D), lambda i: (0, 0)),
            pl.BlockSpec((1, D), lambda i: (0, 0)),
        ],
        out_specs=pl.BlockSpec((NCORE, _R, H), lambda i: (0, i, 0)),
        out_shape=jax.ShapeDtypeStruct((NCORE, N, H), jnp.float32),
    )(summed, xs, cnt, A, B, bias)


def _tc2(summed, h1, cnt, A, B, bias):
    D2 = 512
    return pl.pallas_call(
        _tc2_body,
        grid=(N // _R,),
        in_specs=[
            pl.BlockSpec((NCORE, _R, H), lambda i: (0, i, 0)),
            pl.BlockSpec((NCORE, _R, H), lambda i: (0, i, 0)),
            pl.BlockSpec((_R, 1), lambda i: (i, 0)),
            pl.BlockSpec((D, D2), lambda i: (0, 0)),
            pl.BlockSpec((D, D2), lambda i: (0, 0)),
            pl.BlockSpec((1, D2), lambda i: (0, 0)),
        ],
        out_specs=pl.BlockSpec((_R, D2), lambda i: (i, 0)),
        out_shape=jax.ShapeDtypeStruct((N, D2), jnp.float32),
    )(summed, h1, cnt, A, B, bias)


def kernel(x, unused, edge_index, W1l, b1, W1r, W2l, b2, W2r):
    xs = x.reshape(N, NCORE, H).transpose(1, 0, 2)
    src = edge_index[0].astype(jnp.int32)
    dst = edge_index[1].astype(jnp.int32)
    srcp = jnp.pad(src, (0, EPAD - E))
    dstp = jnp.pad(dst, (0, EPAD - E), constant_values=N)
    src2 = jnp.concatenate([srcp, srcp + N])
    zf = jnp.zeros((RPT, H), jnp.float32)
    zc = jnp.zeros((RPT, 16), jnp.float32)
    ones = jnp.ones((CH, 16), jnp.float32)

    seg1 = _make_segsum(True)
    seg2 = _make_segsum(False)

    sum1, cnt16 = seg1(xs.reshape(NCORE * N, H), src2, dstp, zf, zc, ones)
    cnt = cnt16[:, :1]

    h1 = _tc1(sum1.reshape(NCORE, N, H), xs, cnt,
              W1l.T, W1r.T, b1[None, :])

    (sum2,) = seg2(h1.reshape(NCORE * N, H), src2, dstp, zf)

    out = _tc2(sum2.reshape(NCORE, N, H), h1, cnt,
               W2l.T, W2r.T, b2[None, :])
    return out


# speedup vs baseline: 2.4360x; 2.4360x over previous
import jax
import jax.numpy as jnp
from jax import lax
from jax.experimental import pallas as pl
from jax.experimental.pallas import tpu as pltpu
from jax.experimental.pallas import tpu_sc as plsc

N = 10000
E = 160000
D = 256
H = 128
NCORE = 2
NSUB = 16
CH = 128
EPT = 10240
EPAD = EPT * NSUB
NCHUNK = EPT // CH
NP = 10112
RPT = NP // NSUB


_PIECES = [(0, CH), (CH, CH), (2 * CH, CH), (3 * CH, CH), (4 * CH, RPT - 4 * CH)]


def _segsum_body(x_hbm, src_hbm, dst_hbm, zf_hbm, sum_hbm,
                 src_v, dst_v, rows_v, acc_sh, sem):
    c = lax.axis_index("c")
    s = lax.axis_index("s")

    pltpu.sync_copy(zf_hbm, rows_v)
    for off, ln in _PIECES:
        pltpu.sync_copy(rows_v.at[pl.ds(0, ln)],
                        acc_sh.at[pl.ds(s * RPT + off, ln)])
    plsc.subcore_barrier()

    for k in range(NCHUNK):
        e0 = c * EPAD + s * EPT + k * CH
        pltpu.sync_copy(src_hbm.at[pl.ds(e0, CH)], src_v)
        pltpu.sync_copy(dst_hbm.at[pl.ds(s * EPT + k * CH, CH)], dst_v)
        pltpu.async_copy(x_hbm.at[src_v], rows_v, sem).wait()
        pltpu.sync_copy(rows_v, acc_sh.at[dst_v], add=True)

    plsc.subcore_barrier()

    for off, ln in _PIECES:
        pltpu.sync_copy(acc_sh.at[pl.ds(s * RPT + off, ln)],
                        rows_v.at[pl.ds(0, ln)])
        pltpu.sync_copy(rows_v.at[pl.ds(0, ln)],
                        sum_hbm.at[pl.ds(c * NP + s * RPT + off, ln)])


def _make_segsum():
    mesh = plsc.VectorSubcoreMesh(core_axis_name="c", subcore_axis_name="s")
    return pl.kernel(
        _segsum_body,
        out_type=jax.ShapeDtypeStruct((NCORE * NP, H), jnp.float32),
        mesh=mesh,
        scratch_types=[
            pltpu.VMEM((CH,), jnp.int32),
            pltpu.VMEM((CH,), jnp.int32),
            pltpu.VMEM((CH, H), jnp.float32),
            pltpu.VMEM_SHARED((NP, H), jnp.float32),
            pltpu.SemaphoreType.DMA,
        ],
    )


ECT = EPAD // (NCORE * NSUB)
NCCH = ECT // CH


def _count_body(dst_hbm, zf_hbm, ones_hbm, cnt_hbm,
                dst_v, ones_v, cnt_sh):
    c = lax.axis_index("c")
    s = lax.axis_index("s")

    pltpu.sync_copy(zf_hbm, ones_v)
    for off, ln in _PIECES:
        pltpu.sync_copy(ones_v.at[pl.ds(0, ln)],
                        cnt_sh.at[pl.ds(s * RPT + off, ln)])
    pltpu.sync_copy(ones_hbm, ones_v)
    plsc.subcore_barrier()

    for k in range(NCCH):
        e0 = (c * NSUB + s) * ECT + k * CH
        pltpu.sync_copy(dst_hbm.at[pl.ds(e0, CH)], dst_v)
        pltpu.sync_copy(ones_v, cnt_sh.at[dst_v], add=True)

    plsc.subcore_barrier()

    for off, ln in _PIECES:
        pltpu.sync_copy(cnt_sh.at[pl.ds(s * RPT + off, ln)],
                        ones_v.at[pl.ds(0, ln)])
        pltpu.sync_copy(ones_v.at[pl.ds(0, ln)],
                        cnt_hbm.at[pl.ds(c * NP + s * RPT + off, ln)])


def _make_count():
    mesh = plsc.VectorSubcoreMesh(core_axis_name="c", subcore_axis_name="s")
    return pl.kernel(
        _count_body,
        out_type=jax.ShapeDtypeStruct((NCORE * NP, H), jnp.float32),
        mesh=mesh,
        scratch_types=[
            pltpu.VMEM((CH,), jnp.int32),
            pltpu.VMEM((CH, H), jnp.float32),
            pltpu.VMEM_SHARED((NP, H), jnp.float32),
        ],
    )


def _tc1_body(sum_ref, x_ref, ca_ref, cb_ref, a_ref, b_ref, bias_ref, out_ref):
    inv = 1.0 / jnp.maximum(ca_ref[...] + cb_ref[...], 1.0)
    m = jnp.concatenate([sum_ref[0], sum_ref[1]], axis=1) * inv
    xc = jnp.concatenate([x_ref[0], x_ref[1]], axis=1)
    h = (jnp.dot(m, a_ref[...], preferred_element_type=jnp.float32)
         + jnp.dot(xc, b_ref[...], preferred_element_type=jnp.float32)
         + bias_ref[...])
    h = jnp.maximum(h, 0.0)
    out_ref[0] = h[:, :H]
    out_ref[1] = h[:, H:]


def _tc2_body(sum_ref, h_ref, ca_ref, cb_ref, a_ref, b_ref, bias_ref, out_ref):
    inv = 1.0 / jnp.maximum(ca_ref[...] + cb_ref[...], 1.0)
    m = jnp.concatenate([sum_ref[0], sum_ref[1]], axis=1) * inv
    hc = jnp.concatenate([h_ref[0], h_ref[1]], axis=1)
    g = (jnp.dot(m, a_ref[...], preferred_element_type=jnp.float32)
         + jnp.dot(hc, b_ref[...], preferred_element_type=jnp.float32)
         + bias_ref[...])
    g = jnp.maximum(g, 0.0)
    ss = jnp.sum(g * g, axis=1, keepdims=True)
    out_ref[...] = g / jnp.maximum(jnp.sqrt(ss), 1e-12)


_R = 1000


def _tc1(summed, xs, cnta, cntb, A, B, bias):
    return pl.pallas_call(
        _tc1_body,
        grid=(N // _R,),
        in_specs=[
            pl.BlockSpec((NCORE, _R, H), lambda i: (0, i, 0)),
            pl.BlockSpec((NCORE, _R, H), lambda i: (0, i, 0)),
            pl.BlockSpec((_R, 1), lambda i: (i, 0)),
            pl.BlockSpec((_R, 1), lambda i: (i, 0)),
            pl.BlockSpec((D, D), lambda i: (0, 0)),
            pl.BlockSpec((D, D), lambda i: (0, 0)),
            pl.BlockSpec((1, D), lambda i: (0, 0)),
        ],
        out_specs=pl.BlockSpec((NCORE, _R, H), lambda i: (0, i, 0)),
        out_shape=jax.ShapeDtypeStruct((NCORE, N, H), jnp.float32),
    )(summed, xs, cnta, cntb, A, B, bias)


def _tc2(summed, h1, cnta, cntb, A, B, bias):
    D2 = 512
    return pl.pallas_call(
        _tc2_body,
        grid=(N // _R,),
        in_specs=[
            pl.BlockSpec((NCORE, _R, H), lambda i: (0, i, 0)),
            pl.BlockSpec((NCORE, _R, H), lambda i: (0, i, 0)),
            pl.BlockSpec((_R, 1), lambda i: (i, 0)),
            pl.BlockSpec((_R, 1), lambda i: (i, 0)),
            pl.BlockSpec((D, D2), lambda i: (0, 0)),
            pl.BlockSpec((D, D2), lambda i: (0, 0)),
            pl.BlockSpec((1, D2), lambda i: (0, 0)),
        ],
        out_specs=pl.BlockSpec((_R, D2), lambda i: (i, 0)),
        out_shape=jax.ShapeDtypeStruct((N, D2), jnp.float32),
    )(summed, h1, cnta, cntb, A, B, bias)


def kernel(x, unused, edge_index, W1l, b1, W1r, W2l, b2, W2r):
    xs = x.reshape(N, NCORE, H).transpose(1, 0, 2)
    src = edge_index[0].astype(jnp.int32)
    dst = edge_index[1].astype(jnp.int32)
    srcp = jnp.pad(src, (0, EPAD - E))
    dstp = jnp.pad(dst, (0, EPAD - E), constant_values=N)
    src2 = jnp.concatenate([srcp, srcp + N])
    zf = jnp.zeros((CH, H), jnp.float32)
    onesf = jnp.ones((CH, H), jnp.float32)

    seg = _make_segsum()
    count = _make_count()

    cnt2 = count(dstp, zf, onesf)
    cnta = cnt2[:NP, :1]
    cntb = cnt2[NP:, :1]

    sum1 = seg(xs.reshape(NCORE * N, H), src2, dstp, zf)

    h1 = _tc1(sum1.reshape(NCORE, NP, H), xs, cnta, cntb,
              W1l.T, W1r.T, b1[None, :])

    sum2 = seg(h1.reshape(NCORE * N, H), src2, dstp, zf)

    out = _tc2(sum2.reshape(NCORE, NP, H), h1, cnta, cntb,
               W2l.T, W2r.T, b2[None, :])
    return out


# speedup vs baseline: 3.1091x; 1.2763x over previous
import jax
import jax.numpy as jnp
from jax import lax
from jax.experimental import pallas as pl
from jax.experimental.pallas import tpu as pltpu
from jax.experimental.pallas import tpu_sc as plsc

N = 10000
E = 160000
D = 256
H = 128
NCORE = 2
NSUB = 16
CH = 128
EPT = 10240
EPAD = EPT * NSUB
NCHUNK = EPT // CH
NP = 10112
RPT = NP // NSUB


_PIECES = [(0, CH), (CH, CH), (2 * CH, CH), (3 * CH, CH), (4 * CH, RPT - 4 * CH)]


NBUF = 3


def _segsum_body(x_hbm, src_hbm, dst_hbm, zf_hbm, sum_hbm,
                 dst_v, s0, s1, s2, r0, r1, r2, acc_sh, m0, m1, m2):
    c = lax.axis_index("c")
    s = lax.axis_index("s")
    srcs = (s0, s1, s2)
    rows = (r0, r1, r2)
    sems = (m0, m1, m2)

    pltpu.sync_copy(zf_hbm, r0)
    for off, ln in _PIECES:
        pltpu.sync_copy(r0.at[pl.ds(0, ln)],
                        acc_sh.at[pl.ds(s * RPT + off, ln)])
    plsc.subcore_barrier()

    base_src = c * EPAD + s * EPT
    base_dst = s * EPT
    handles = [None] * NCHUNK

    def issue(k):
        b = k % NBUF
        pltpu.sync_copy(src_hbm.at[pl.ds(base_src + k * CH, CH)], srcs[b])
        handles[k] = pltpu.async_copy(x_hbm.at[srcs[b]], rows[b], sems[b])

    for k in range(NBUF - 1):
        issue(k)
    for k in range(NCHUNK):
        if k + NBUF - 1 < NCHUNK:
            issue(k + NBUF - 1)
        handles[k].wait()
        pltpu.sync_copy(dst_hbm.at[pl.ds(base_dst + k * CH, CH)], dst_v)
        pltpu.sync_copy(rows[k % NBUF], acc_sh.at[dst_v], add=True)

    plsc.subcore_barrier()

    for off, ln in _PIECES:
        pltpu.sync_copy(acc_sh.at[pl.ds(s * RPT + off, ln)],
                        r0.at[pl.ds(0, ln)])
        pltpu.sync_copy(r0.at[pl.ds(0, ln)],
                        sum_hbm.at[pl.ds(c * NP + s * RPT + off, ln)])


def _make_segsum():
    mesh = plsc.VectorSubcoreMesh(core_axis_name="c", subcore_axis_name="s")
    return pl.kernel(
        _segsum_body,
        out_type=jax.ShapeDtypeStruct((NCORE * NP, H), jnp.float32),
        mesh=mesh,
        scratch_types=[
            pltpu.VMEM((CH,), jnp.int32),
            pltpu.VMEM((CH,), jnp.int32),
            pltpu.VMEM((CH,), jnp.int32),
            pltpu.VMEM((CH,), jnp.int32),
            pltpu.VMEM((CH, H), jnp.float32),
            pltpu.VMEM((CH, H), jnp.float32),
            pltpu.VMEM((CH, H), jnp.float32),
            pltpu.VMEM_SHARED((NP, H), jnp.float32),
            pltpu.SemaphoreType.DMA,
            pltpu.SemaphoreType.DMA,
            pltpu.SemaphoreType.DMA,
        ],
    )


ECT = EPAD // (NCORE * NSUB)
NCCH = ECT // CH


def _count_body(dst_hbm, zf_hbm, ones_hbm, cnt_hbm,
                dst_v, ones_v, cnt_sh):
    c = lax.axis_index("c")
    s = lax.axis_index("s")

    pltpu.sync_copy(zf_hbm, ones_v)
    for off, ln in _PIECES:
        pltpu.sync_copy(ones_v.at[pl.ds(0, ln)],
                        cnt_sh.at[pl.ds(s * RPT + off, ln)])
    pltpu.sync_copy(ones_hbm, ones_v)
    plsc.subcore_barrier()

    for k in range(NCCH):
        e0 = (c * NSUB + s) * ECT + k * CH
        pltpu.sync_copy(dst_hbm.at[pl.ds(e0, CH)], dst_v)
        pltpu.sync_copy(ones_v, cnt_sh.at[dst_v], add=True)

    plsc.subcore_barrier()

    for off, ln in _PIECES:
        pltpu.sync_copy(cnt_sh.at[pl.ds(s * RPT + off, ln)],
                        ones_v.at[pl.ds(0, ln)])
        pltpu.sync_copy(ones_v.at[pl.ds(0, ln)],
                        cnt_hbm.at[pl.ds(c * NP + s * RPT + off, ln)])


def _make_count():
    mesh = plsc.VectorSubcoreMesh(core_axis_name="c", subcore_axis_name="s")
    return pl.kernel(
        _count_body,
        out_type=jax.ShapeDtypeStruct((NCORE * NP, H), jnp.float32),
        mesh=mesh,
        scratch_types=[
            pltpu.VMEM((CH,), jnp.int32),
            pltpu.VMEM((CH, H), jnp.float32),
            pltpu.VMEM_SHARED((NP, H), jnp.float32),
        ],
    )


def _tc1_body(sum_ref, x_ref, ca_ref, cb_ref, a_ref, b_ref, bias_ref, out_ref):
    inv = 1.0 / jnp.maximum(ca_ref[...] + cb_ref[...], 1.0)
    m = jnp.concatenate([sum_ref[0], sum_ref[1]], axis=1) * inv
    xc = jnp.concatenate([x_ref[0], x_ref[1]], axis=1)
    h = (jnp.dot(m, a_ref[...], preferred_element_type=jnp.float32)
         + jnp.dot(xc, b_ref[...], preferred_element_type=jnp.float32)
         + bias_ref[...])
    h = jnp.maximum(h, 0.0)
    out_ref[0] = h[:, :H]
    out_ref[1] = h[:, H:]


def _tc2_body(sum_ref, h_ref, ca_ref, cb_ref, a_ref, b_ref, bias_ref, out_ref):
    inv = 1.0 / jnp.maximum(ca_ref[...] + cb_ref[...], 1.0)
    m = jnp.concatenate([sum_ref[0], sum_ref[1]], axis=1) * inv
    hc = jnp.concatenate([h_ref[0], h_ref[1]], axis=1)
    g = (jnp.dot(m, a_ref[...], preferred_element_type=jnp.float32)
         + jnp.dot(hc, b_ref[...], preferred_element_type=jnp.float32)
         + bias_ref[...])
    g = jnp.maximum(g, 0.0)
    ss = jnp.sum(g * g, axis=1, keepdims=True)
    out_ref[...] = g / jnp.maximum(jnp.sqrt(ss), 1e-12)


_R = 1000


def _tc1(summed, xs, cnta, cntb, A, B, bias):
    return pl.pallas_call(
        _tc1_body,
        grid=(N // _R,),
        in_specs=[
            pl.BlockSpec((NCORE, _R, H), lambda i: (0, i, 0)),
            pl.BlockSpec((NCORE, _R, H), lambda i: (0, i, 0)),
            pl.BlockSpec((_R, 1), lambda i: (i, 0)),
            pl.BlockSpec((_R, 1), lambda i: (i, 0)),
            pl.BlockSpec((D, D), lambda i: (0, 0)),
            pl.BlockSpec((D, D), lambda i: (0, 0)),
            pl.BlockSpec((1, D), lambda i: (0, 0)),
        ],
        out_specs=pl.BlockSpec((NCORE, _R, H), lambda i: (0, i, 0)),
        out_shape=jax.ShapeDtypeStruct((NCORE, N, H), jnp.float32),
    )(summed, xs, cnta, cntb, A, B, bias)


def _tc2(summed, h1, cnta, cntb, A, B, bias):
    D2 = 512
    return pl.pallas_call(
        _tc2_body,
        grid=(N // _R,),
        in_specs=[
            pl.BlockSpec((NCORE, _R, H), lambda i: (0, i, 0)),
            pl.BlockSpec((NCORE, _R, H), lambda i: (0, i, 0)),
            pl.BlockSpec((_R, 1), lambda i: (i, 0)),
            pl.BlockSpec((_R, 1), lambda i: (i, 0)),
            pl.BlockSpec((D, D2), lambda i: (0, 0)),
            pl.BlockSpec((D, D2), lambda i: (0, 0)),
            pl.BlockSpec((1, D2), lambda i: (0, 0)),
        ],
        out_specs=pl.BlockSpec((_R, D2), lambda i: (i, 0)),
        out_shape=jax.ShapeDtypeStruct((N, D2), jnp.float32),
    )(summed, h1, cnta, cntb, A, B, bias)


def kernel(x, unused, edge_index, W1l, b1, W1r, W2l, b2, W2r):
    xs = x.reshape(N, NCORE, H).transpose(1, 0, 2)
    src = edge_index[0].astype(jnp.int32)
    dst = edge_index[1].astype(jnp.int32)
    srcp = jnp.pad(src, (0, EPAD - E))
    dstp = jnp.pad(dst, (0, EPAD - E), constant_values=N)
    src2 = jnp.concatenate([srcp, srcp + N])
    zf = jnp.zeros((CH, H), jnp.float32)
    onesf = jnp.ones((CH, H), jnp.float32)

    seg = _make_segsum()
    count = _make_count()

    cnt2 = count(dstp, zf, onesf)
    cnta = cnt2[:NP, :1]
    cntb = cnt2[NP:, :1]

    sum1 = seg(xs.reshape(NCORE * N, H), src2, dstp, zf)

    h1 = _tc1(sum1.reshape(NCORE, NP, H), xs, cnta, cntb,
              W1l.T, W1r.T, b1[None, :])

    sum2 = seg(h1.reshape(NCORE * N, H), src2, dstp, zf)

    out = _tc2(sum2.reshape(NCORE, NP, H), h1, cnta, cntb,
               W2l.T, W2r.T, b2[None, :])
    return out


# speedup vs baseline: 3.2167x; 1.0346x over previous
import jax
import jax.numpy as jnp
from jax import lax
from jax.experimental import pallas as pl
from jax.experimental.pallas import tpu as pltpu
from jax.experimental.pallas import tpu_sc as plsc

N = 10000
E = 160000
D = 256
H = 128
NCORE = 2
NSUB = 16
CH = 128
EPT = 10240
EPAD = EPT * NSUB
NCHUNK = EPT // CH
NP = 10112
RPT = NP // NSUB


_PIECES = [(0, CH), (CH, CH), (2 * CH, CH), (3 * CH, CH), (4 * CH, RPT - 4 * CH)]


NBUF = 3


def _segsum_body(x_hbm, src_hbm, dst_hbm, zf_hbm, sum_hbm,
                 d0, d1, d2, s0, s1, s2, r0, r1, r2, acc_sh,
                 m0, m1, m2, n0, n1, n2):
    c = lax.axis_index("c")
    s = lax.axis_index("s")
    srcs = (s0, s1, s2)
    dsts = (d0, d1, d2)
    rows = (r0, r1, r2)
    sems = (m0, m1, m2)
    dsems = (n0, n1, n2)

    pltpu.sync_copy(zf_hbm, r0)
    for off, ln in _PIECES:
        pltpu.sync_copy(r0.at[pl.ds(0, ln)],
                        acc_sh.at[pl.ds(s * RPT + off, ln)])
    plsc.subcore_barrier()

    base_src = c * EPAD + s * EPT
    base_dst = s * EPT
    handles = [None] * NCHUNK
    dhandles = [None] * NCHUNK

    def issue(k):
        b = k % NBUF
        pltpu.sync_copy(src_hbm.at[pl.ds(base_src + k * CH, CH)], srcs[b])
        handles[k] = pltpu.async_copy(x_hbm.at[srcs[b]], rows[b], sems[b])
        dhandles[k] = pltpu.async_copy(
            dst_hbm.at[pl.ds(base_dst + k * CH, CH)], dsts[b], dsems[b])

    for k in range(NBUF - 1):
        issue(k)
    for k in range(NCHUNK):
        if k + NBUF - 1 < NCHUNK:
            issue(k + NBUF - 1)
        b = k % NBUF
        handles[k].wait()
        dhandles[k].wait()
        pltpu.sync_copy(rows[b], acc_sh.at[dsts[b]], add=True)

    plsc.subcore_barrier()

    for off, ln in _PIECES:
        pltpu.sync_copy(acc_sh.at[pl.ds(s * RPT + off, ln)],
                        r0.at[pl.ds(0, ln)])
        pltpu.sync_copy(r0.at[pl.ds(0, ln)],
                        sum_hbm.at[pl.ds(c * NP + s * RPT + off, ln)])


def _make_segsum():
    mesh = plsc.VectorSubcoreMesh(core_axis_name="c", subcore_axis_name="s")
    return pl.kernel(
        _segsum_body,
        out_type=jax.ShapeDtypeStruct((NCORE * NP, H), jnp.float32),
        mesh=mesh,
        scratch_types=[
            pltpu.VMEM((CH,), jnp.int32),
            pltpu.VMEM((CH,), jnp.int32),
            pltpu.VMEM((CH,), jnp.int32),
            pltpu.VMEM((CH,), jnp.int32),
            pltpu.VMEM((CH,), jnp.int32),
            pltpu.VMEM((CH,), jnp.int32),
            pltpu.VMEM((CH, H), jnp.float32),
            pltpu.VMEM((CH, H), jnp.float32),
            pltpu.VMEM((CH, H), jnp.float32),
            pltpu.VMEM_SHARED((NP, H), jnp.float32),
            pltpu.SemaphoreType.DMA,
            pltpu.SemaphoreType.DMA,
            pltpu.SemaphoreType.DMA,
            pltpu.SemaphoreType.DMA,
            pltpu.SemaphoreType.DMA,
            pltpu.SemaphoreType.DMA,
        ],
    )


ECT = EPAD // (NCORE * NSUB)
NCCH = ECT // CH


def _count_body(dst_hbm, zf_hbm, ones_hbm, cnt_hbm,
                dst_v, ones_v, cnt_sh):
    c = lax.axis_index("c")
    s = lax.axis_index("s")

    pltpu.sync_copy(zf_hbm, ones_v)
    for off, ln in _PIECES:
        pltpu.sync_copy(ones_v.at[pl.ds(0, ln)],
                        cnt_sh.at[pl.ds(s * RPT + off, ln)])
    pltpu.sync_copy(ones_hbm, ones_v)
    plsc.subcore_barrier()

    for k in range(NCCH):
        e0 = (c * NSUB + s) * ECT + k * CH
        pltpu.sync_copy(dst_hbm.at[pl.ds(e0, CH)], dst_v)
        pltpu.sync_copy(ones_v, cnt_sh.at[dst_v], add=True)

    plsc.subcore_barrier()

    for off, ln in _PIECES:
        pltpu.sync_copy(cnt_sh.at[pl.ds(s * RPT + off, ln)],
                        ones_v.at[pl.ds(0, ln)])
        pltpu.sync_copy(ones_v.at[pl.ds(0, ln)],
                        cnt_hbm.at[pl.ds(c * NP + s * RPT + off, ln)])


def _make_count():
    mesh = plsc.VectorSubcoreMesh(core_axis_name="c", subcore_axis_name="s")
    return pl.kernel(
        _count_body,
        out_type=jax.ShapeDtypeStruct((NCORE * NP, H), jnp.float32),
        mesh=mesh,
        scratch_types=[
            pltpu.VMEM((CH,), jnp.int32),
            pltpu.VMEM((CH, H), jnp.float32),
            pltpu.VMEM_SHARED((NP, H), jnp.float32),
        ],
    )


def _tc1_body(sum_ref, x_ref, ca_ref, cb_ref, a_ref, b_ref, bias_ref, out_ref):
    inv = 1.0 / jnp.maximum(ca_ref[...] + cb_ref[...], 1.0)
    m = jnp.concatenate([sum_ref[0], sum_ref[1]], axis=1) * inv
    xc = jnp.concatenate([x_ref[0], x_ref[1]], axis=1)
    h = (jnp.dot(m, a_ref[...], preferred_element_type=jnp.float32)
         + jnp.dot(xc, b_ref[...], preferred_element_type=jnp.float32)
         + bias_ref[...])
    h = jnp.maximum(h, 0.0)
    out_ref[0] = h[:, :H]
    out_ref[1] = h[:, H:]


def _tc2_body(sum_ref, h_ref, ca_ref, cb_ref, a_ref, b_ref, bias_ref, out_ref):
    inv = 1.0 / jnp.maximum(ca_ref[...] + cb_ref[...], 1.0)
    m = jnp.concatenate([sum_ref[0], sum_ref[1]], axis=1) * inv
    hc = jnp.concatenate([h_ref[0], h_ref[1]], axis=1)
    g = (jnp.dot(m, a_ref[...], preferred_element_type=jnp.float32)
         + jnp.dot(hc, b_ref[...], preferred_element_type=jnp.float32)
         + bias_ref[...])
    g = jnp.maximum(g, 0.0)
    ss = jnp.sum(g * g, axis=1, keepdims=True)
    out_ref[...] = g / jnp.maximum(jnp.sqrt(ss), 1e-12)


_R = 1000


def _tc1(summed, xs, cnta, cntb, A, B, bias):
    return pl.pallas_call(
        _tc1_body,
        grid=(N // _R,),
        in_specs=[
            pl.BlockSpec((NCORE, _R, H), lambda i: (0, i, 0)),
            pl.BlockSpec((NCORE, _R, H), lambda i: (0, i, 0)),
            pl.BlockSpec((_R, 1), lambda i: (i, 0)),
            pl.BlockSpec((_R, 1), lambda i: (i, 0)),
            pl.BlockSpec((D, D), lambda i: (0, 0)),
            pl.BlockSpec((D, D), lambda i: (0, 0)),
            pl.BlockSpec((1, D), lambda i: (0, 0)),
        ],
        out_specs=pl.BlockSpec((NCORE, _R, H), lambda i: (0, i, 0)),
        out_shape=jax.ShapeDtypeStruct((NCORE, N, H), jnp.float32),
    )(summed, xs, cnta, cntb, A, B, bias)


def _tc2(summed, h1, cnta, cntb, A, B, bias):
    D2 = 512
    return pl.pallas_call(
        _tc2_body,
        grid=(N // _R,),
        in_specs=[
            pl.BlockSpec((NCORE, _R, H), lambda i: (0, i, 0)),
            pl.BlockSpec((NCORE, _R, H), lambda i: (0, i, 0)),
            pl.BlockSpec((_R, 1), lambda i: (i, 0)),
            pl.BlockSpec((_R, 1), lambda i: (i, 0)),
            pl.BlockSpec((D, D2), lambda i: (0, 0)),
            pl.BlockSpec((D, D2), lambda i: (0, 0)),
            pl.BlockSpec((1, D2), lambda i: (0, 0)),
        ],
        out_specs=pl.BlockSpec((_R, D2), lambda i: (i, 0)),
        out_shape=jax.ShapeDtypeStruct((N, D2), jnp.float32),
    )(summed, h1, cnta, cntb, A, B, bias)


def kernel(x, unused, edge_index, W1l, b1, W1r, W2l, b2, W2r):
    xs = x.reshape(N, NCORE, H).transpose(1, 0, 2)
    src = edge_index[0].astype(jnp.int32)
    dst = edge_index[1].astype(jnp.int32)
    srcp = jnp.pad(src, (0, EPAD - E))
    dstp = jnp.pad(dst, (0, EPAD - E), constant_values=N)
    src2 = jnp.concatenate([srcp, srcp + N])
    zf = jnp.zeros((CH, H), jnp.float32)
    onesf = jnp.ones((CH, H), jnp.float32)

    seg = _make_segsum()
    count = _make_count()

    cnt2 = count(dstp, zf, onesf)
    cnta = cnt2[:NP, :1]
    cntb = cnt2[NP:, :1]

    sum1 = seg(xs.reshape(NCORE * N, H), src2, dstp, zf)

    h1 = _tc1(sum1.reshape(NCORE, NP, H), xs, cnta, cntb,
              W1l.T, W1r.T, b1[None, :])

    sum2 = seg(h1.reshape(NCORE * N, H), src2, dstp, zf)

    out = _tc2(sum2.reshape(NCORE, NP, H), h1, cnta, cntb,
               W2l.T, W2r.T, b2[None, :])
    return out
